# baseline (speedup 1.0000x reference)
import jax
import jax.numpy as jnp
from jax import lax
from jax.experimental import pallas as pl
from jax.experimental.pallas import tpu as pltpu
from jax.experimental.pallas import tpu_sc as plsc

TOKENS = 32768
ACTORS = 32768
DM = 2048
NCH = 16

ROW_BLK = 2048
NWORKERS = 32
BPW = ACTORS // NWORKERS


def _proj_body(x_ref, w_ref, b_ref, lp_ref):
    logits = jnp.dot(x_ref[...], w_ref[...], preferred_element_type=jnp.float32)
    lp_ref[...] = logits + b_ref[...]


def _project(x, W, b2):
    return pl.pallas_call(
        _proj_body,
        grid=(TOKENS // ROW_BLK,),
        in_specs=[
            pl.BlockSpec((ROW_BLK, DM), lambda i: (i, 0)),
            pl.BlockSpec((DM, NCH), lambda i: (0, 0)),
            pl.BlockSpec((1, NCH), lambda i: (0, 0)),
        ],
        out_specs=pl.BlockSpec((ROW_BLK, NCH), lambda i: (i, 0)),
        out_shape=jax.ShapeDtypeStruct((TOKENS, NCH), jnp.float32),
        compiler_params=pltpu.CompilerParams(
            vmem_limit_bytes=100 * 1024 * 1024),
    )(x, W, b2)


def _sc_body(lp_hbm, actors_hbm, out_lp, idx_v, rows_v, sem):
    ncores = lax.axis_size("c")
    wid = lax.axis_index("s") * ncores + lax.axis_index("c")
    base = wid * BPW
    pltpu.sync_copy(actors_hbm.at[pl.ds(base, BPW)], idx_v)
    pltpu.async_copy(lp_hbm.at[idx_v], rows_v, sem).wait()
    pltpu.sync_copy(rows_v, out_lp.at[pl.ds(base, BPW)])


def _sc_gather(lp_all, actors):
    mesh = plsc.VectorSubcoreMesh(core_axis_name="c", subcore_axis_name="s")
    k = pl.kernel(
        _sc_body,
        out_type=jax.ShapeDtypeStruct((ACTORS, NCH), jnp.float32),
        mesh=mesh,
        compiler_params=pltpu.CompilerParams(use_tc_tiling_on_sc=False),
        scratch_types=[
            pltpu.VMEM((BPW,), jnp.int32),
            pltpu.VMEM((BPW, NCH), jnp.float32),
            pltpu.SemaphoreType.DMA,
        ],
    )
    return k(lp_all, actors)


def _epi_body(lg_ref, pa_ref, lp_ref, logp_ref, ent_ref):
    logits = lg_ref[...]
    m = jnp.max(logits, axis=-1, keepdims=True)
    e = jnp.exp(logits - m)
    lse = m + jnp.log(jnp.sum(e, axis=-1, keepdims=True))
    lp = logits - lse
    lp_ref[...] = lp
    cols = lax.broadcasted_iota(jnp.int32, (ROW_BLK, NCH), 1)
    sel = cols == pa_ref[...]
    logp_ref[...] = jnp.sum(jnp.where(sel, lp, 0.0), axis=-1, keepdims=True)
    ent_ref[...] = -jnp.sum(jnp.exp(lp) * lp, axis=-1, keepdims=True)


def _epilogue(logits_g, pa2):
    return pl.pallas_call(
        _epi_body,
        grid=(ACTORS // ROW_BLK,),
        in_specs=[
            pl.BlockSpec((ROW_BLK, NCH), lambda i: (i, 0)),
            pl.BlockSpec((ROW_BLK, 1), lambda i: (i, 0)),
        ],
        out_specs=[
            pl.BlockSpec((ROW_BLK, NCH), lambda i: (i, 0)),
            pl.BlockSpec((ROW_BLK, 1), lambda i: (i, 0)),
            pl.BlockSpec((ROW_BLK, 1), lambda i: (i, 0)),
        ],
        out_shape=[
            jax.ShapeDtypeStruct((ACTORS, NCH), jnp.float32),
            jax.ShapeDtypeStruct((ACTORS, 1), jnp.float32),
            jax.ShapeDtypeStruct((ACTORS, 1), jnp.float32),
        ],
    )(logits_g, pa2)


def kernel(x, actors, lengths, prev_actions, W, b):
    logits_all = _project(x, W, b.reshape(1, NCH))
    logits_g = _sc_gather(logits_all, actors)
    log_probs, logprob, entropy = _epilogue(
        logits_g, prev_actions.reshape(ACTORS, 1))
    return (prev_actions, lengths, logprob.reshape(ACTORS),
            entropy.reshape(ACTORS), log_probs)


# speedup vs baseline: 1.0286x; 1.0286x over previous
import jax
import jax.numpy as jnp
from jax import lax
from jax.experimental import pallas as pl
from jax.experimental.pallas import tpu as pltpu
from jax.experimental.pallas import tpu_sc as plsc

TOKENS = 32768
ACTORS = 32768
DM = 2048
NCH = 16

ROW_BLK = 2048
NWORKERS = 32
BPW = ACTORS // NWORKERS


def _proj_body(x_ref, w_ref, b_ref, lp_ref):
    logits = jnp.dot(x_ref[...], w_ref[...], preferred_element_type=jnp.float32)
    logits = logits + b_ref[...]
    m = jnp.max(logits, axis=-1, keepdims=True)
    e = jnp.exp(logits - m)
    lse = m + jnp.log(jnp.sum(e, axis=-1, keepdims=True))
    lp_ref[...] = logits - lse


def _project(x, W, b2):
    return pl.pallas_call(
        _proj_body,
        grid=(TOKENS // ROW_BLK,),
        in_specs=[
            pl.BlockSpec((ROW_BLK, DM), lambda i: (i, 0)),
            pl.BlockSpec((DM, NCH), lambda i: (0, 0)),
            pl.BlockSpec((1, NCH), lambda i: (0, 0)),
        ],
        out_specs=pl.BlockSpec((ROW_BLK, NCH), lambda i: (i, 0)),
        out_shape=jax.ShapeDtypeStruct((TOKENS, NCH), jnp.float32),
        compiler_params=pltpu.CompilerParams(
            vmem_limit_bytes=100 * 1024 * 1024),
    )(x, W, b2)


def _sc_body(lp_hbm, actors_hbm, out_lp, idx_v, rows_v, sem):
    ncores = lax.axis_size("c")
    wid = lax.axis_index("s") * ncores + lax.axis_index("c")
    base = wid * BPW
    pltpu.sync_copy(actors_hbm.at[pl.ds(base, BPW)], idx_v)
    pltpu.async_copy(lp_hbm.at[idx_v], rows_v, sem).wait()
    pltpu.sync_copy(rows_v, out_lp.at[pl.ds(base, BPW)])


def _sc_gather(lp_all, actors):
    mesh = plsc.VectorSubcoreMesh(core_axis_name="c", subcore_axis_name="s")
    k = pl.kernel(
        _sc_body,
        out_type=jax.ShapeDtypeStruct((ACTORS, NCH), jnp.float32),
        mesh=mesh,
        compiler_params=pltpu.CompilerParams(use_tc_tiling_on_sc=False),
        scratch_types=[
            pltpu.VMEM((BPW,), jnp.int32),
            pltpu.VMEM((BPW, NCH), jnp.float32),
            pltpu.SemaphoreType.DMA,
        ],
    )
    return k(lp_all, actors)


def _epi_body(lp_g_ref, pa_ref, logp_ref, ent_ref):
    lp = lp_g_ref[...]
    cols = lax.broadcasted_iota(jnp.int32, (ROW_BLK, NCH), 1)
    sel = cols == pa_ref[...]
    logp_ref[...] = jnp.sum(jnp.where(sel, lp, 0.0), axis=-1, keepdims=True)
    ent_ref[...] = -jnp.sum(jnp.exp(lp) * lp, axis=-1, keepdims=True)


def _epilogue(log_probs, pa2):
    return pl.pallas_call(
        _epi_body,
        grid=(ACTORS // ROW_BLK,),
        in_specs=[
            pl.BlockSpec((ROW_BLK, NCH), lambda i: (i, 0)),
            pl.BlockSpec((ROW_BLK, 1), lambda i: (i, 0)),
        ],
        out_specs=[
            pl.BlockSpec((ROW_BLK, 1), lambda i: (i, 0)),
            pl.BlockSpec((ROW_BLK, 1), lambda i: (i, 0)),
        ],
        out_shape=[
            jax.ShapeDtypeStruct((ACTORS, 1), jnp.float32),
            jax.ShapeDtypeStruct((ACTORS, 1), jnp.float32),
        ],
    )(log_probs, pa2)


def kernel(x, actors, lengths, prev_actions, W, b):
    lp_all = _project(x, W, b.reshape(1, NCH))
    log_probs = _sc_gather(lp_all, actors)
    logprob, entropy = _epilogue(log_probs, prev_actions.reshape(ACTORS, 1))
    return (prev_actions, lengths, logprob.reshape(ACTORS),
            entropy.reshape(ACTORS), log_probs)
